# baseline (device time: 42870 ns/iter reference)
import jax
import jax.numpy as jnp
from jax import lax
from jax.experimental import pallas as pl
from jax.experimental.pallas import tpu as pltpu

N_DEV = 8
NBUF = 4


def kernel(x, w_mat):
    m_per, k = x.shape
    n = w_mat.shape[1]
    n_per = n // N_DEV

    def body(x_hbm, w_hbm, out_hbm, xf_ref, xb_ref, wslab_ref, wb_ref,
             y_ref, rb_ref, of_ref, xdma_sem, wdma_sems, odma_sems,
             send_sems, recv_sems):
        my = lax.axis_index("i")

        barrier = pltpu.get_barrier_semaphore()
        for d in range(1, N_DEV):
            pl.semaphore_signal(
                barrier, inc=1,
                device_id=((my + d) % N_DEV,),
                device_id_type=pl.DeviceIdType.MESH,
            )

        def wslab_copy(d):
            t = (my + d) % N_DEV
            return pltpu.make_async_copy(
                w_hbm.at[:, pl.ds(t * n_per, n_per)],
                wslab_ref.at[d % NBUF],
                wdma_sems.at[d % NBUF],
            )

        def out_copy(s, slot):
            return pltpu.make_async_copy(
                of_ref.at[slot],
                out_hbm.at[pl.ds(s * m_per, m_per), :],
                odma_sems.at[slot],
            )

        xcopy = pltpu.make_async_copy(x_hbm, xf_ref, xdma_sem)
        xcopy.start()
        for d in range(NBUF):
            wslab_copy(d).start()

        xcopy.wait()
        xb_ref[:, :] = xf_ref[:, :].astype(jnp.bfloat16)

        wslab_copy(0).wait()
        wb_ref[0, :, :] = wslab_ref[0].astype(jnp.bfloat16)

        for d in range(N_DEV):
            t = (my + d) % N_DEV
            if d + 1 < N_DEV:
                wslab_copy(d + 1).wait()
                wb_ref[(d + 1) % 2, :, :] = (
                    wslab_ref[(d + 1) % NBUF].astype(jnp.bfloat16))
            blk = jnp.dot(xb_ref[:, :], wb_ref[d % 2],
                          preferred_element_type=jnp.float32)
            if d + NBUF < N_DEV:
                wslab_copy(d + NBUF).start()

            if d == 0:
                of_ref[0, :, :] = blk
                out_copy(my, 0).start()
                pl.semaphore_wait(barrier, N_DEV - 1)
            else:
                y_ref[d, :, :] = blk.astype(jnp.bfloat16)
                rdma = pltpu.make_async_remote_copy(
                    src_ref=y_ref.at[d],
                    dst_ref=rb_ref.at[my],
                    send_sem=send_sems.at[d],
                    recv_sem=recv_sems.at[my],
                    device_id=(t,),
                    device_id_type=pl.DeviceIdType.MESH,
                )
                rdma.start()

        for d in range(1, N_DEV):
            s = (my - d) % N_DEV
            slot = d % 2
            desc = pltpu.make_async_remote_copy(
                src_ref=y_ref.at[d],
                dst_ref=rb_ref.at[s],
                send_sem=send_sems.at[d],
                recv_sem=recv_sems.at[s],
                device_id=(s,),
                device_id_type=pl.DeviceIdType.MESH,
            )
            desc.wait_recv()
            if d >= 2:
                out_copy((my - (d - 2)) % N_DEV, slot).wait()
            of_ref[slot, :, :] = rb_ref[s].astype(jnp.float32)
            out_copy(s, slot).start()

        out_copy((my - 6) % N_DEV, 0).wait()
        out_copy((my - 7) % N_DEV, 1).wait()

        for d in range(1, N_DEV):
            desc = pltpu.make_async_remote_copy(
                src_ref=y_ref.at[d],
                dst_ref=rb_ref.at[my],
                send_sem=send_sems.at[d],
                recv_sem=recv_sems.at[my],
                device_id=((my + d) % N_DEV,),
                device_id_type=pl.DeviceIdType.MESH,
            )
            desc.wait_send()

    return pl.pallas_call(
        body,
        out_shape=jax.ShapeDtypeStruct((N_DEV * m_per, n_per), jnp.float32),
        in_specs=[
            pl.BlockSpec(memory_space=pltpu.MemorySpace.HBM),
            pl.BlockSpec(memory_space=pltpu.MemorySpace.HBM),
        ],
        out_specs=pl.BlockSpec(memory_space=pltpu.MemorySpace.HBM),
        scratch_shapes=[
            pltpu.VMEM((m_per, k), jnp.float32),
            pltpu.VMEM((m_per, k), jnp.bfloat16),
            pltpu.VMEM((NBUF, k, n_per), jnp.float32),
            pltpu.VMEM((2, k, n_per), jnp.bfloat16),
            pltpu.VMEM((N_DEV, m_per, n_per), jnp.bfloat16),
            pltpu.VMEM((N_DEV, m_per, n_per), jnp.bfloat16),
            pltpu.VMEM((2, m_per, n_per), jnp.float32),
            pltpu.SemaphoreType.DMA,
            pltpu.SemaphoreType.DMA((NBUF,)),
            pltpu.SemaphoreType.DMA((2,)),
            pltpu.SemaphoreType.DMA((N_DEV,)),
            pltpu.SemaphoreType.DMA((N_DEV,)),
        ],
        compiler_params=pltpu.CompilerParams(
            collective_id=0,
            vmem_limit_bytes=100 * 1024 * 1024,
        ),
    )(x, w_mat)


# device time: 35142 ns/iter; 1.2199x vs baseline; 1.2199x over previous
import jax
import jax.numpy as jnp
from jax import lax
from jax.experimental import pallas as pl
from jax.experimental.pallas import tpu as pltpu

N_DEV = 8
MS = 2


def kernel(x, w_mat):
    m_per, k = x.shape
    n = w_mat.shape[1]
    n_per = n // N_DEV
    mh = m_per // MS

    def body(x_ref, w_hbm, out_ref, xb_ref, wslab_ref, y_ref, rb_ref,
             wdma_sems, send_sems, recv_sems):
        my = lax.axis_index("i")

        barrier = pltpu.get_barrier_semaphore()
        for d in range(1, N_DEV):
            pl.semaphore_signal(
                barrier, inc=1,
                device_id=((my + d) % N_DEV,),
                device_id_type=pl.DeviceIdType.MESH,
            )

        def wslab_copy(d):
            t = (my + d) % N_DEV
            return pltpu.make_async_copy(
                w_hbm.at[:, pl.ds(t * n_per, n_per)],
                wslab_ref.at[d % 2],
                wdma_sems.at[d % 2],
            )

        wslab_copy(0).start()
        wslab_copy(1).start()

        xb_ref[:, :] = x_ref[:, :].astype(jnp.bfloat16)

        for d in range(N_DEV):
            t = (my + d) % N_DEV
            wslab_copy(d).wait()
            wb = wslab_ref[d % 2].astype(jnp.bfloat16)
            for h in range(MS):
                blk = jnp.dot(xb_ref[pl.ds(h * mh, mh), :], wb,
                              preferred_element_type=jnp.float32)
                if d == 0:
                    out_ref[pl.ds(my * m_per + h * mh, mh), :] = blk
                else:
                    y_ref[d, pl.ds(h * mh, mh), :] = blk.astype(jnp.bfloat16)
                    rdma = pltpu.make_async_remote_copy(
                        src_ref=y_ref.at[d, pl.ds(h * mh, mh), :],
                        dst_ref=rb_ref.at[my, pl.ds(h * mh, mh), :],
                        send_sem=send_sems.at[d * MS + h],
                        recv_sem=recv_sems.at[my * MS + h],
                        device_id=(t,),
                        device_id_type=pl.DeviceIdType.MESH,
                    )
                    rdma.start()
            if d == 0:
                pl.semaphore_wait(barrier, N_DEV - 1)
            if d + 2 < N_DEV:
                wslab_copy(d + 2).start()

        for d in range(1, N_DEV):
            s = (my - d) % N_DEV
            for h in range(MS):
                desc = pltpu.make_async_remote_copy(
                    src_ref=y_ref.at[d, pl.ds(h * mh, mh), :],
                    dst_ref=rb_ref.at[s, pl.ds(h * mh, mh), :],
                    send_sem=send_sems.at[d * MS + h],
                    recv_sem=recv_sems.at[s * MS + h],
                    device_id=(s,),
                    device_id_type=pl.DeviceIdType.MESH,
                )
                desc.wait_recv()
            out_ref[pl.ds(s * m_per, m_per), :] = rb_ref[s].astype(jnp.float32)

        for d in range(1, N_DEV):
            for h in range(MS):
                desc = pltpu.make_async_remote_copy(
                    src_ref=y_ref.at[d, pl.ds(h * mh, mh), :],
                    dst_ref=rb_ref.at[my, pl.ds(h * mh, mh), :],
                    send_sem=send_sems.at[d * MS + h],
                    recv_sem=recv_sems.at[my * MS + h],
                    device_id=((my + d) % N_DEV,),
                    device_id_type=pl.DeviceIdType.MESH,
                )
                desc.wait_send()

    return pl.pallas_call(
        body,
        out_shape=jax.ShapeDtypeStruct((N_DEV * m_per, n_per), jnp.float32),
        in_specs=[
            pl.BlockSpec(memory_space=pltpu.VMEM),
            pl.BlockSpec(memory_space=pltpu.MemorySpace.HBM),
        ],
        out_specs=pl.BlockSpec(memory_space=pltpu.VMEM),
        scratch_shapes=[
            pltpu.VMEM((m_per, k), jnp.bfloat16),
            pltpu.VMEM((2, k, n_per), jnp.float32),
            pltpu.VMEM((N_DEV, m_per, n_per), jnp.bfloat16),
            pltpu.VMEM((N_DEV, m_per, n_per), jnp.bfloat16),
            pltpu.SemaphoreType.DMA((2,)),
            pltpu.SemaphoreType.DMA((N_DEV * MS,)),
            pltpu.SemaphoreType.DMA((N_DEV * MS,)),
        ],
        compiler_params=pltpu.CompilerParams(
            collective_id=0,
            vmem_limit_bytes=100 * 1024 * 1024,
        ),
    )(x, w_mat)


# device time: 34200 ns/iter; 1.2535x vs baseline; 1.0275x over previous
import jax
import jax.numpy as jnp
from jax import lax
from jax.experimental import pallas as pl
from jax.experimental.pallas import tpu as pltpu

N_DEV = 8
MS = 2


def kernel(x, w_mat):
    m_per, k = x.shape
    n = w_mat.shape[1]
    n_per = n // N_DEV
    mh = m_per // MS
    seq = list(range(1, N_DEV)) + [0]

    def body(x_ref, w_hbm, out_ref, xb_ref, wslab_ref, y_ref, rb_ref,
             wdma_sems, send_sems, recv_sems):
        my = lax.axis_index("i")

        barrier = pltpu.get_barrier_semaphore()
        for d in range(1, N_DEV):
            pl.semaphore_signal(
                barrier, inc=1,
                device_id=((my + d) % N_DEV,),
                device_id_type=pl.DeviceIdType.MESH,
            )

        def wslab_copy(i):
            t = (my + seq[i]) % N_DEV
            return pltpu.make_async_copy(
                w_hbm.at[:, pl.ds(t * n_per, n_per)],
                wslab_ref.at[i % 2],
                wdma_sems.at[i % 2],
            )

        wslab_copy(0).start()
        wslab_copy(1).start()

        xb_ref[:, :] = x_ref[:, :].astype(jnp.bfloat16)

        for i, d in enumerate(seq):
            t = (my + d) % N_DEV
            wslab_copy(i).wait()
            wb = wslab_ref[i % 2].astype(jnp.bfloat16)
            for h in range(MS):
                blk = jnp.dot(xb_ref[pl.ds(h * mh, mh), :], wb,
                              preferred_element_type=jnp.float32)
                if d == 0:
                    out_ref[pl.ds(my * m_per + h * mh, mh), :] = blk
                else:
                    y_ref[d, pl.ds(h * mh, mh), :] = blk.astype(jnp.bfloat16)
                    if i == 0 and h == 0:
                        pl.semaphore_wait(barrier, N_DEV - 1)
                    rdma = pltpu.make_async_remote_copy(
                        src_ref=y_ref.at[d, pl.ds(h * mh, mh), :],
                        dst_ref=rb_ref.at[my, pl.ds(h * mh, mh), :],
                        send_sem=send_sems.at[d * MS + h],
                        recv_sem=recv_sems.at[my * MS + h],
                        device_id=(t,),
                        device_id_type=pl.DeviceIdType.MESH,
                    )
                    rdma.start()
            if i + 2 < N_DEV:
                wslab_copy(i + 2).start()

        for d in range(1, N_DEV):
            s = (my - d) % N_DEV
            for h in range(MS):
                desc = pltpu.make_async_remote_copy(
                    src_ref=y_ref.at[d, pl.ds(h * mh, mh), :],
                    dst_ref=rb_ref.at[s, pl.ds(h * mh, mh), :],
                    send_sem=send_sems.at[d * MS + h],
                    recv_sem=recv_sems.at[s * MS + h],
                    device_id=(s,),
                    device_id_type=pl.DeviceIdType.MESH,
                )
                desc.wait_recv()
            out_ref[pl.ds(s * m_per, m_per), :] = rb_ref[s].astype(jnp.float32)

        for d in range(1, N_DEV):
            for h in range(MS):
                desc = pltpu.make_async_remote_copy(
                    src_ref=y_ref.at[d, pl.ds(h * mh, mh), :],
                    dst_ref=rb_ref.at[my, pl.ds(h * mh, mh), :],
                    send_sem=send_sems.at[d * MS + h],
                    recv_sem=recv_sems.at[my * MS + h],
                    device_id=((my + d) % N_DEV,),
                    device_id_type=pl.DeviceIdType.MESH,
                )
                desc.wait_send()

    return pl.pallas_call(
        body,
        out_shape=jax.ShapeDtypeStruct((N_DEV * m_per, n_per), jnp.float32),
        in_specs=[
            pl.BlockSpec(memory_space=pltpu.VMEM),
            pl.BlockSpec(memory_space=pltpu.MemorySpace.HBM),
        ],
        out_specs=pl.BlockSpec(memory_space=pltpu.VMEM),
        scratch_shapes=[
            pltpu.VMEM((m_per, k), jnp.bfloat16),
            pltpu.VMEM((2, k, n_per), jnp.float32),
            pltpu.VMEM((N_DEV, m_per, n_per), jnp.bfloat16),
            pltpu.VMEM((N_DEV, m_per, n_per), jnp.bfloat16),
            pltpu.SemaphoreType.DMA((2,)),
            pltpu.SemaphoreType.DMA((N_DEV * MS,)),
            pltpu.SemaphoreType.DMA((N_DEV * MS,)),
        ],
        compiler_params=pltpu.CompilerParams(
            collective_id=0,
            vmem_limit_bytes=100 * 1024 * 1024,
        ),
    )(x, w_mat)
